# Initial kernel scaffold; baseline (speedup 1.0000x reference)
import jax
import jax.numpy as jnp
from jax.experimental import pallas as pl


def kernel(dydx_flat):
    raise NotImplementedError("write your pallas kernel here")


# speedup vs baseline: 1.2376x; 1.2376x over previous
import jax
import jax.numpy as jnp
from jax.experimental import pallas as pl
from jax.experimental.pallas import tpu as pltpu

_LANE = 128
_TILE_B = 8
_SUB_CHUNK = 16


def _gp_tile_kernel(x_ref, out_ref):
    r = x_ref.shape[1]
    acc = jnp.zeros((x_ref.shape[0], _SUB_CHUNK, _LANE), jnp.float32)
    for j in range(0, r, _SUB_CHUNK):
        blk = x_ref[:, j : j + _SUB_CHUNK, :].astype(jnp.float32)
        acc = acc + blk * blk
    col = jnp.sum(acc, axis=1)
    ssq = jnp.sum(col, axis=-1, keepdims=True)
    out_ref[...] = (jnp.sqrt(ssq) - 1.0) ** 2


def _gradient_penalty(x):
    B, F = x.shape
    R = -(-F // _LANE)
    B_pad = -(-B // _TILE_B) * _TILE_B
    F_pad = R * _LANE
    if (B_pad, F_pad) != (B, F):
        x = jnp.pad(x, ((0, B_pad - B), (0, F_pad - F)))
    x3 = x.reshape(B_pad, R, _LANE)

    per_sample = pl.pallas_call(
        _gp_tile_kernel,
        out_shape=jax.ShapeDtypeStruct((B_pad, 1), jnp.float32),
        grid=(B_pad // _TILE_B,),
        in_specs=[pl.BlockSpec((_TILE_B, R, _LANE), lambda i: (i, 0, 0))],
        out_specs=pl.BlockSpec((_TILE_B, 1), lambda i: (i, 0)),
        compiler_params=pltpu.CompilerParams(
            dimension_semantics=("parallel",),
            vmem_limit_bytes=64 * 1024 * 1024,
        ),
    )(x3)

    return jnp.mean(per_sample[:B, 0])


def kernel(dydx_flat):
    return _gradient_penalty(dydx_flat)


# speedup vs baseline: 4.1851x; 3.3817x over previous
import jax
import jax.numpy as jnp
from jax.experimental import pallas as pl
from jax.experimental.pallas import tpu as pltpu

_LANE = 128
_TILE_B = 8
_CHUNK = 2048


def _gp_tile_kernel(x_ref, out_ref):
    f = x_ref.shape[1]
    acc = jnp.zeros((x_ref.shape[0], _CHUNK), jnp.float32)
    for j in range(0, f, _CHUNK):
        blk = x_ref[:, j : j + _CHUNK].astype(jnp.float32)
        acc = acc + blk * blk
    ssq = jnp.sum(acc, axis=-1, keepdims=True)
    out_ref[...] = (jnp.sqrt(ssq) - 1.0) ** 2


def _gradient_penalty(x):
    B, F = x.shape
    B_pad = -(-B // _TILE_B) * _TILE_B
    F_pad = -(-F // _CHUNK) * _CHUNK
    if (B_pad, F_pad) != (B, F):
        x = jnp.pad(x, ((0, B_pad - B), (0, F_pad - F)))

    per_sample = pl.pallas_call(
        _gp_tile_kernel,
        out_shape=jax.ShapeDtypeStruct((B_pad, 1), jnp.float32),
        grid=(B_pad // _TILE_B,),
        in_specs=[pl.BlockSpec((_TILE_B, F_pad), lambda i: (i, 0))],
        out_specs=pl.BlockSpec((_TILE_B, 1), lambda i: (i, 0)),
        compiler_params=pltpu.CompilerParams(
            dimension_semantics=("parallel",),
            vmem_limit_bytes=64 * 1024 * 1024,
        ),
    )(x)

    return jnp.mean(per_sample[:B, 0])


def kernel(dydx_flat):
    return _gradient_penalty(dydx_flat)


# speedup vs baseline: 4.5795x; 1.0942x over previous
import jax
import jax.numpy as jnp
from jax.experimental import pallas as pl
from jax.experimental.pallas import tpu as pltpu

_LANE = 128
_TILE_B = 16
_CHUNK = 2048


def _gp_tile_kernel(x_ref, out_ref):
    f = x_ref.shape[1]
    acc = jnp.zeros((x_ref.shape[0], _CHUNK), jnp.float32)
    for j in range(0, f, _CHUNK):
        blk = x_ref[:, j : j + _CHUNK].astype(jnp.float32)
        acc = acc + blk * blk
    ssq = jnp.sum(acc, axis=-1, keepdims=True)
    out_ref[...] = (jnp.sqrt(ssq) - 1.0) ** 2


def _gradient_penalty(x):
    B, F = x.shape
    B_pad = -(-B // _TILE_B) * _TILE_B
    F_pad = -(-F // _CHUNK) * _CHUNK
    if (B_pad, F_pad) != (B, F):
        x = jnp.pad(x, ((0, B_pad - B), (0, F_pad - F)))

    per_sample = pl.pallas_call(
        _gp_tile_kernel,
        out_shape=jax.ShapeDtypeStruct((B_pad, 1), jnp.float32),
        grid=(B_pad // _TILE_B,),
        in_specs=[pl.BlockSpec((_TILE_B, F_pad), lambda i: (i, 0))],
        out_specs=pl.BlockSpec((_TILE_B, 1), lambda i: (i, 0)),
        compiler_params=pltpu.CompilerParams(
            dimension_semantics=("parallel",),
            vmem_limit_bytes=64 * 1024 * 1024,
        ),
    )(x)

    return jnp.mean(per_sample[:B, 0])


def kernel(dydx_flat):
    return _gradient_penalty(dydx_flat)
